# baseline (speedup 1.0000x reference)
import functools

import jax
import jax.numpy as jnp
from jax import lax
from jax.experimental import pallas as pl
from jax.experimental.pallas import tpu as pltpu
from jax.experimental.pallas import tpu_sc as plsc

B = 16384
NF = 26
V = 100000
D = 64
DP = 128
NG = 10
L = 16

NC = 2
NS = 16
NW = NC * NS
BW = B // NW
BC = 16
NCHUNK = BW // BC


def _body(fidx, gw, tab, emb, out, idx_v, s_v, w_v, e_v, gsem):
    wid = lax.axis_index("s") * NC + lax.axis_index("c")
    pltpu.sync_copy(emb, e_v)

    def chunk(c, carry):
        b0 = wid * BW + c * BC
        pltpu.sync_copy(fidx.at[pl.ds(b0, BC), :], idx_v)
        cps = [
            pltpu.async_copy(
                tab.at[idx_v.at[b]], s_v.at[b, pl.ds(0, NF), :], gsem
            )
            for b in range(BC)
        ]
        pltpu.sync_copy(gw.at[pl.ds(b0, BC), :], w_v)
        for b in range(BC):
            wv = w_v[b, :]
            ws = [wv[g] for g in range(NG)]
            q = ws[0]
            for g in range(1, NG):
                q = q + ws[g]
            qi = 1.0 / jnp.broadcast_to(q, (L,))
            for k in range(D // L):
                acc = ws[0] * e_v[0, pl.ds(k * L, L)]
                for g in range(1, NG):
                    acc = acc + ws[g] * e_v[g, pl.ds(k * L, L)]
                s_v[b, NF, pl.ds(k * L, L)] = acc * qi
        for cp in cps:
            cp.wait()
        pltpu.sync_copy(s_v, out.at[pl.ds(b0, BC), :, :])
        return carry

    lax.fori_loop(0, NCHUNK, chunk, 0)


@jax.jit
def _embed(fidx, gw, tab, emb):
    mesh = plsc.VectorSubcoreMesh(core_axis_name="c", subcore_axis_name="s")
    kfn = functools.partial(
        pl.kernel,
        mesh=mesh,
        out_type=jax.ShapeDtypeStruct((B, NF + 1, DP), jnp.float32),
        scratch_types=[
            pltpu.VMEM((BC, NF), jnp.int32),
            pltpu.VMEM((BC, NF + 1, DP), jnp.float32),
            pltpu.VMEM((BC, L), jnp.float32),
            pltpu.VMEM((NG, D), jnp.float32),
            pltpu.SemaphoreType.DMA,
        ],
    )(_body)
    return _slice_out(kfn(fidx, gw, tab, emb))


def _pad_body(i_ref, o_ref):
    o_ref[:, pl.ds(0, D)] = i_ref[...]


_PAD_ROWS = 8000


def _pad_table(tab2d):
    grid = (NF * V) // _PAD_ROWS
    return pl.pallas_call(
        _pad_body,
        grid=(grid,),
        in_specs=[pl.BlockSpec((_PAD_ROWS, D), lambda i: (i, 0))],
        out_specs=pl.BlockSpec((_PAD_ROWS, DP), lambda i: (i, 0)),
        out_shape=jax.ShapeDtypeStruct((NF * V, DP), jnp.float32),
    )(tab2d)


_SL_ROWS = 512


def _slice_body(i_ref, o_ref):
    o_ref[...] = i_ref[:, :, pl.ds(0, D)]


def _slice_out(o128):
    grid = B // _SL_ROWS
    return pl.pallas_call(
        _slice_body,
        grid=(grid,),
        in_specs=[pl.BlockSpec((_SL_ROWS, NF + 1, DP), lambda i: (i, 0, 0))],
        out_specs=pl.BlockSpec((_SL_ROWS, NF + 1, D), lambda i: (i, 0, 0)),
        out_shape=jax.ShapeDtypeStruct((B, NF + 1, D), jnp.float32),
    )(o128)


def kernel(x, tables, genre_embed):
    fidx = x[:, :NF] + (jnp.arange(NF, dtype=jnp.int32) * V)[None, :]
    gw = jnp.pad(x[:, NF:].astype(jnp.float32), ((0, 0), (0, L - NG)))
    tab = _pad_table(tables.reshape(NF * V, D))
    return _embed(fidx, gw, tab, genre_embed)


# speedup vs baseline: 1.2290x; 1.2290x over previous
import functools

import jax
import jax.numpy as jnp
from jax import lax
from jax.experimental import pallas as pl
from jax.experimental.pallas import tpu as pltpu
from jax.experimental.pallas import tpu_sc as plsc

B = 16384
NF = 26
V = 100000
D = 64
DP = 128
NG = 10
L = 16

NC = 2
NS = 16
NW = NC * NS
BW = B // NW
BC = 16
NCHUNK = BW // BC


def _body(fidx, gw, tab, emb, out, idx_v, s_v, w_v, e_v, gsem):
    wid = lax.axis_index("s") * NC + lax.axis_index("c")
    pltpu.sync_copy(emb, e_v)

    def chunk(c, carry):
        b0 = wid * BW + c * BC
        pltpu.sync_copy(fidx.at[pl.ds(b0, BC), :], idx_v)
        cps = [
            pltpu.async_copy(
                tab.at[idx_v.at[b]], s_v.at[b, pl.ds(0, NF), :], gsem
            )
            for b in range(BC)
        ]
        pltpu.sync_copy(gw.at[pl.ds(b0, BC), :], w_v)
        for b in range(BC):
            wv = w_v[b, :]
            ws = [wv[g] for g in range(NG)]
            q = ws[0]
            for g in range(1, NG):
                q = q + ws[g]
            qi = 1.0 / jnp.broadcast_to(q, (L,))
            for k in range(D // L):
                acc = ws[0] * e_v[0, pl.ds(k * L, L)]
                for g in range(1, NG):
                    acc = acc + ws[g] * e_v[g, pl.ds(k * L, L)]
                s_v[b, NF, pl.ds(k * L, L)] = acc * qi
        for cp in cps:
            cp.wait()
        pltpu.sync_copy(s_v, out.at[pl.ds(b0, BC), :, :])
        return carry

    lax.fori_loop(0, NCHUNK, chunk, 0)


@jax.jit
def _embed(fidx, gw, tab, emb):
    mesh = plsc.VectorSubcoreMesh(core_axis_name="c", subcore_axis_name="s")
    kfn = functools.partial(
        pl.kernel,
        mesh=mesh,
        out_type=jax.ShapeDtypeStruct((B, NF + 1, DP), jnp.float32),
        scratch_types=[
            pltpu.VMEM((BC, NF), jnp.int32),
            pltpu.VMEM((BC, NF + 1, DP), jnp.float32),
            pltpu.VMEM((BC, L), jnp.float32),
            pltpu.VMEM((NG, D), jnp.float32),
            pltpu.SemaphoreType.DMA,
        ],
    )(_body)
    return kfn(fidx, gw, tab, emb)[:, :, :D]


_TV = 2048


def _tp_body(i_ref, o_ref):
    o_ref[0, :, pl.ds(0, D)] = jnp.transpose(i_ref[0], (1, 0))


def _transpose_pad(tabT):
    gv = (V + _TV - 1) // _TV
    out = pl.pallas_call(
        _tp_body,
        grid=(NF, gv),
        in_specs=[pl.BlockSpec((1, D, _TV), lambda f, j: (f, 0, j))],
        out_specs=pl.BlockSpec((1, _TV, DP), lambda f, j: (f, j, 0)),
        out_shape=jax.ShapeDtypeStruct((NF, V, DP), jnp.float32),
    )(tabT)
    return out.reshape(NF * V, DP)


def kernel(x, tables, genre_embed):
    fidx = x[:, :NF] + (jnp.arange(NF, dtype=jnp.int32) * V)[None, :]
    gw = jnp.pad(x[:, NF:].astype(jnp.float32), ((0, 0), (0, L - NG)))
    tab = _transpose_pad(tables.swapaxes(1, 2))
    return _embed(fidx, gw, tab, genre_embed)


# speedup vs baseline: 1.7896x; 1.4561x over previous
import functools

import jax
import jax.numpy as jnp
from jax import lax
from jax.experimental import pallas as pl
from jax.experimental.pallas import tpu as pltpu
from jax.experimental.pallas import tpu_sc as plsc

B = 16384
NF = 26
V = 100000
D = 64
DP = 128
NG = 10
L = 16

NC = 2
NS = 16
NW = NC * NS
BW = B // NW
BC = 16
NCHUNK = BW // BC


def _body(fidx, gw, tab, emb, out, idx_v, s_v, w_v, e_v, gsem):
    wid = lax.axis_index("s") * NC + lax.axis_index("c")
    pltpu.sync_copy(emb, e_v)

    def chunk(c, carry):
        b0 = wid * BW + c * BC
        pltpu.sync_copy(fidx.at[pl.ds(b0, BC), :], idx_v)
        cps = [
            pltpu.async_copy(
                tab.at[idx_v.at[b]], s_v.at[b, pl.ds(0, NF), :], gsem
            )
            for b in range(BC)
        ]
        pltpu.sync_copy(gw.at[pl.ds(b0, BC), :], w_v)
        for b in range(BC):
            wv = w_v[b, :]
            ws = [wv[g] for g in range(NG)]
            q = ws[0]
            for g in range(1, NG):
                q = q + ws[g]
            qi = 1.0 / jnp.broadcast_to(q, (L,))
            for k in range(D // L):
                acc = ws[0] * e_v[0, pl.ds(k * L, L)]
                for g in range(1, NG):
                    acc = acc + ws[g] * e_v[g, pl.ds(k * L, L)]
                s_v[b, NF, pl.ds(k * L, L)] = acc * qi
        for cp in cps:
            cp.wait()
        pltpu.sync_copy(s_v, out.at[pl.ds(b0, BC), :, :])
        return carry

    lax.fori_loop(0, NCHUNK, chunk, 0)


@jax.jit
def _embed(fidx, gw, tab, emb):
    mesh = plsc.VectorSubcoreMesh(core_axis_name="c", subcore_axis_name="s")
    kfn = functools.partial(
        pl.kernel,
        mesh=mesh,
        out_type=jax.ShapeDtypeStruct((B, NF + 1, DP), jnp.float32),
        scratch_types=[
            pltpu.VMEM((BC, NF), jnp.int32),
            pltpu.VMEM((BC, NF + 1, DP), jnp.float32),
            pltpu.VMEM((BC, L), jnp.float32),
            pltpu.VMEM((NG, D), jnp.float32),
            pltpu.SemaphoreType.DMA,
        ],
    )(_body)
    return kfn(fidx, gw, tab, emb)[:, :, :D]


_TV = 8192


def _tp_body(i_ref, o_ref):
    o_ref[0, :, pl.ds(0, D)] = jnp.transpose(i_ref[0], (1, 0))


def _transpose_pad(tabT):
    gv = (V + _TV - 1) // _TV
    out = pl.pallas_call(
        _tp_body,
        grid=(NF, gv),
        in_specs=[pl.BlockSpec((1, D, _TV), lambda f, j: (f, 0, j))],
        out_specs=pl.BlockSpec((1, _TV, DP), lambda f, j: (f, j, 0)),
        out_shape=jax.ShapeDtypeStruct((NF, V, DP), jnp.float32),
    )(tabT)
    return out.reshape(NF * V, DP)


def kernel(x, tables, genre_embed):
    fidx = x[:, :NF] + (jnp.arange(NF, dtype=jnp.int32) * V)[None, :]
    gw = jnp.pad(x[:, NF:].astype(jnp.float32), ((0, 0), (0, L - NG)))
    tab = _transpose_pad(tables.swapaxes(1, 2))
    return _embed(fidx, gw, tab, genre_embed)


# speedup vs baseline: 1.8784x; 1.0496x over previous
import functools

import jax
import jax.numpy as jnp
from jax import lax
from jax.experimental import pallas as pl
from jax.experimental.pallas import tpu as pltpu
from jax.experimental.pallas import tpu_sc as plsc

B = 16384
NF = 26
V = 100000
D = 64
DP = 128
NG = 10
L = 16

NC = 2
NS = 16
NW = NC * NS
BW = B // NW
BC = 16
NCHUNK = BW // BC


def _body(fidx, gw, tab, emb, out, idx_v, s_v, w_v, e_v, gsem):
    wid = lax.axis_index("s") * NC + lax.axis_index("c")
    pltpu.sync_copy(emb, e_v)

    def chunk(c, carry):
        b0 = wid * BW + c * BC
        pltpu.sync_copy(fidx.at[pl.ds(b0, BC), :], idx_v)
        cps = [
            pltpu.async_copy(
                tab.at[idx_v.at[b]], s_v.at[b, pl.ds(0, NF), :], gsem
            )
            for b in range(BC)
        ]
        pltpu.sync_copy(gw.at[pl.ds(b0, BC), :], w_v)
        for b in range(BC):
            wv = w_v[b, :]
            ws = [wv[g] for g in range(NG)]
            q = ws[0]
            for g in range(1, NG):
                q = q + ws[g]
            qi = 1.0 / jnp.broadcast_to(q, (L,))
            for k in range(D // L):
                acc = ws[0] * e_v[0, pl.ds(k * L, L)]
                for g in range(1, NG):
                    acc = acc + ws[g] * e_v[g, pl.ds(k * L, L)]
                s_v[b, NF, pl.ds(k * L, L)] = acc * qi
        for cp in cps:
            cp.wait()
        pltpu.sync_copy(s_v, out.at[pl.ds(b0, BC), :, :])
        return carry

    lax.fori_loop(0, NCHUNK, chunk, 0)


@jax.jit
def _embed(fidx, gw, tab, emb):
    mesh = plsc.VectorSubcoreMesh(core_axis_name="c", subcore_axis_name="s")
    kfn = functools.partial(
        pl.kernel,
        mesh=mesh,
        out_type=jax.ShapeDtypeStruct((B, NF + 1, DP), jnp.float32),
        scratch_types=[
            pltpu.VMEM((BC, NF), jnp.int32),
            pltpu.VMEM((BC, NF + 1, DP), jnp.float32),
            pltpu.VMEM((BC, L), jnp.float32),
            pltpu.VMEM((NG, D), jnp.float32),
            pltpu.SemaphoreType.DMA,
        ],
    )(_body)
    return kfn(fidx, gw, tab, emb)[:, :, :D]


_TV = 16384


def _tp_body(i_ref, o_ref):
    o_ref[0, :, pl.ds(0, D)] = jnp.transpose(i_ref[0], (1, 0))


def _transpose_pad(tabT):
    gv = (V + _TV - 1) // _TV
    out = pl.pallas_call(
        _tp_body,
        grid=(NF, gv),
        in_specs=[pl.BlockSpec((1, D, _TV), lambda f, j: (f, 0, j))],
        out_specs=pl.BlockSpec((1, _TV, DP), lambda f, j: (f, j, 0)),
        out_shape=jax.ShapeDtypeStruct((NF, V, DP), jnp.float32),
    )(tabT)
    return out.reshape(NF * V, DP)


def kernel(x, tables, genre_embed):
    fidx = x[:, :NF] + (jnp.arange(NF, dtype=jnp.int32) * V)[None, :]
    gw = jnp.pad(x[:, NF:].astype(jnp.float32), ((0, 0), (0, L - NG)))
    tab = _transpose_pad(tables.swapaxes(1, 2))
    return _embed(fidx, gw, tab, genre_embed)


# speedup vs baseline: 1.9126x; 1.0182x over previous
import functools

import jax
import jax.numpy as jnp
from jax import lax
from jax.experimental import pallas as pl
from jax.experimental.pallas import tpu as pltpu
from jax.experimental.pallas import tpu_sc as plsc

B = 16384
NF = 26
V = 100000
D = 64
DP = 128
NG = 10
L = 16

NC = 2
NS = 16
NW = NC * NS
BW = B // NW
BC = 16
NCHUNK = BW // BC


def _body(fidx, gw, tab, emb, out, idx_v, s_v, w_v, e_v, gsem):
    wid = lax.axis_index("s") * NC + lax.axis_index("c")
    pltpu.sync_copy(emb, e_v)

    def chunk(c, carry):
        b0 = wid * BW + c * BC
        pltpu.sync_copy(fidx.at[pl.ds(b0, BC), :], idx_v)
        cps = [
            pltpu.async_copy(
                tab.at[idx_v.at[b]], s_v.at[b, pl.ds(0, NF), :], gsem
            )
            for b in range(BC)
        ]
        pltpu.sync_copy(gw.at[pl.ds(b0, BC), :], w_v)
        for b in range(BC):
            wv = w_v[b, :]
            ws = [wv[g] for g in range(NG)]
            q = ws[0]
            for g in range(1, NG):
                q = q + ws[g]
            qi = 1.0 / jnp.broadcast_to(q, (L,))
            for k in range(D // L):
                acc = ws[0] * e_v[0, pl.ds(k * L, L)]
                for g in range(1, NG):
                    acc = acc + ws[g] * e_v[g, pl.ds(k * L, L)]
                s_v[b, NF, pl.ds(k * L, L)] = acc * qi
        for cp in cps:
            cp.wait()
        pltpu.sync_copy(s_v, out.at[pl.ds(b0, BC), :, :])
        return carry

    lax.fori_loop(0, NCHUNK, chunk, 0)


@jax.jit
def _embed(fidx, gw, tab, emb):
    mesh = plsc.VectorSubcoreMesh(core_axis_name="c", subcore_axis_name="s")
    kfn = functools.partial(
        pl.kernel,
        mesh=mesh,
        out_type=jax.ShapeDtypeStruct((B, NF + 1, DP), jnp.float32),
        scratch_types=[
            pltpu.VMEM((BC, NF), jnp.int32),
            pltpu.VMEM((BC, NF + 1, DP), jnp.float32),
            pltpu.VMEM((BC, L), jnp.float32),
            pltpu.VMEM((NG, D), jnp.float32),
            pltpu.SemaphoreType.DMA,
        ],
    )(_body)
    return kfn(fidx, gw, tab, emb)[:, :, :D]


_TV = 32768


def _tp_body(i_ref, o_ref):
    o_ref[0, :, pl.ds(0, D)] = jnp.transpose(i_ref[0], (1, 0))


def _transpose_pad(tabT):
    gv = (V + _TV - 1) // _TV
    out = pl.pallas_call(
        _tp_body,
        grid=(NF, gv),
        in_specs=[pl.BlockSpec((1, D, _TV), lambda f, j: (f, 0, j))],
        out_specs=pl.BlockSpec((1, _TV, DP), lambda f, j: (f, j, 0)),
        out_shape=jax.ShapeDtypeStruct((NF, V, DP), jnp.float32),
    )(tabT)
    return out.reshape(NF * V, DP)


def kernel(x, tables, genre_embed):
    fidx = x[:, :NF] + (jnp.arange(NF, dtype=jnp.int32) * V)[None, :]
    gw = jnp.pad(x[:, NF:].astype(jnp.float32), ((0, 0), (0, L - NG)))
    tab = _transpose_pad(tables.swapaxes(1, 2))
    return _embed(fidx, gw, tab, genre_embed)


# speedup vs baseline: 1.9644x; 1.0271x over previous
import functools

import jax
import jax.numpy as jnp
from jax import lax
from jax.experimental import pallas as pl
from jax.experimental.pallas import tpu as pltpu
from jax.experimental.pallas import tpu_sc as plsc

B = 16384
NF = 26
V = 100000
D = 64
DP = 128
NG = 10
L = 16

NC = 2
NS = 16
NW = NC * NS
BW = B // NW
BC = 8
NCHUNK = BW // BC


def _body(fidx, gw, tab, emb, out, idx_a, idx_b, s_a, s_b, w_a, w_b, e_v,
          gsem, osem):
    wid = lax.axis_index("s") * NC + lax.axis_index("c")
    pltpu.sync_copy(emb, e_v)
    base = wid * BW
    pltpu.sync_copy(fidx.at[pl.ds(base, BC), :], idx_a)
    pltpu.sync_copy(gw.at[pl.ds(base, BC), :], w_a)

    def half(c, bufs, first, last):
        idx_v, s_v, w_v, idx_n, w_n = bufs
        b0 = base + c * BC
        cps = [
            pltpu.async_copy(
                tab.at[idx_v.at[b]], s_v.at[b, pl.ds(0, NF), :], gsem
            )
            for b in range(BC)
        ]

        @pl.when(c + 1 < NCHUNK)
        def _():
            bn = base + (c + 1) * BC
            pltpu.sync_copy(fidx.at[pl.ds(bn, BC), :], idx_n)
            pltpu.sync_copy(gw.at[pl.ds(bn, BC), :], w_n)

        for b in range(BC):
            wv = w_v[b, :]
            ws = [wv[g] for g in range(NG)]
            q = ws[0]
            for g in range(1, NG):
                q = q + ws[g]
            qi = 1.0 / jnp.broadcast_to(q, (L,))
            for k in range(D // L):
                acc = ws[0] * e_v[0, pl.ds(k * L, L)]
                for g in range(1, NG):
                    acc = acc + ws[g] * e_v[g, pl.ds(k * L, L)]
                s_v[b, NF, pl.ds(k * L, L)] = acc * qi
        for cp in cps:
            cp.wait()

        @pl.when(jnp.logical_not(first))
        def _():
            pltpu.make_async_copy(s_b if s_v is s_a else s_a,
                                  out.at[pl.ds(b0 - BC, BC), :, :], osem).wait()
        ocp = pltpu.async_copy(s_v, out.at[pl.ds(b0, BC), :, :], osem)

        @pl.when(last)
        def _():
            ocp.wait()

    def pair(i, carry):
        c = i * 2
        half(c, (idx_a, s_a, w_a, idx_b, w_b), i == 0, jnp.bool_(False))
        half(c + 1, (idx_b, s_b, w_b, idx_a, w_a), jnp.bool_(False),
             i == NCHUNK // 2 - 1)
        return carry

    lax.fori_loop(0, NCHUNK // 2, pair, 0)


@jax.jit
def _embed(fidx, gw, tab, emb):
    mesh = plsc.VectorSubcoreMesh(core_axis_name="c", subcore_axis_name="s")
    kfn = functools.partial(
        pl.kernel,
        mesh=mesh,
        out_type=jax.ShapeDtypeStruct((B, NF + 1, DP), jnp.float32),
        scratch_types=[
            pltpu.VMEM((BC, NF), jnp.int32),
            pltpu.VMEM((BC, NF), jnp.int32),
            pltpu.VMEM((BC, NF + 1, DP), jnp.float32),
            pltpu.VMEM((BC, NF + 1, DP), jnp.float32),
            pltpu.VMEM((BC, L), jnp.float32),
            pltpu.VMEM((BC, L), jnp.float32),
            pltpu.VMEM((NG, D), jnp.float32),
            pltpu.SemaphoreType.DMA,
            pltpu.SemaphoreType.DMA,
        ],
    )(_body)
    return kfn(fidx, gw, tab, emb)[:, :, :D]


_TV = 32768


def _tp_body(i_ref, o_ref):
    o_ref[0, :, pl.ds(0, D)] = jnp.transpose(i_ref[0], (1, 0))


def _transpose_pad(tabT):
    gv = (V + _TV - 1) // _TV
    out = pl.pallas_call(
        _tp_body,
        grid=(NF, gv),
        in_specs=[pl.BlockSpec((1, D, _TV), lambda f, j: (f, 0, j))],
        out_specs=pl.BlockSpec((1, _TV, DP), lambda f, j: (f, j, 0)),
        out_shape=jax.ShapeDtypeStruct((NF, V, DP), jnp.float32),
    )(tabT)
    return out.reshape(NF * V, DP)


def kernel(x, tables, genre_embed):
    fidx = x[:, :NF] + (jnp.arange(NF, dtype=jnp.int32) * V)[None, :]
    gw = jnp.pad(x[:, NF:].astype(jnp.float32), ((0, 0), (0, L - NG)))
    tab = _transpose_pad(tables.swapaxes(1, 2))
    return _embed(fidx, gw, tab, genre_embed)
